# baseline (speedup 1.0000x reference)
import functools

import jax
import jax.numpy as jnp
from jax import lax
from jax.experimental import pallas as pl
from jax.experimental.pallas import tpu as pltpu
from jax.experimental.pallas import tpu_sc as plsc

BETA = 0.4
B = 16384
N_F = 32
L = 16
NC, NS = 2, 16
NW = NC * NS
BPW = B // NW
NCK = BPW // L
NSEG = 4
SEG = BPW // NSEG
W = 8192

_LN2 = 0.6931471805599453


def _detile_body(nrows, *refs):
    half = len(refs) // 2
    ins, outs = refs[:half], refs[half:]
    for ref, oref, nr in zip(ins, outs, nrows):
        x = ref[...]
        if nr < N_F:
            x = jnp.concatenate(
                [x, jnp.zeros((N_F - nr, W), jnp.float32)], axis=0)
        t3 = x.T.reshape(W // 4, 4, N_F)
        oref[...] = jnp.concatenate([t3[:, a, :] for a in range(4)], axis=1)


def _detile(arrays_t):
    n = arrays_t[0].shape[1]
    nrows = tuple(a.shape[0] for a in arrays_t)
    no = n * N_F // 128
    return pl.pallas_call(
        functools.partial(_detile_body, nrows),
        grid=(pl.cdiv(n, W),),
        in_specs=[pl.BlockSpec((nr, W), lambda c: (0, c)) for nr in nrows],
        out_specs=[pl.BlockSpec((W // 4, 128), lambda c: (c, 0))
                   for _ in arrays_t],
        out_shape=[jax.ShapeDtypeStruct((no, 128), jnp.float32)
                   for _ in arrays_t],
    )(*arrays_t)


def _dev_t(d):
    ad = jnp.abs(d)
    bits = lax.bitcast_convert_type(ad, jnp.int32)
    e = (bits >> 23) - 127
    m = lax.bitcast_convert_type((bits & 0x7FFFFF) | 0x3F800000, jnp.float32)
    z = (m - 1.0) / (m + 1.0)
    z2 = z * z
    lnm = 2.0 * z * (1.0 + z2 * (1.0 / 3.0 + z2 * (0.2 + z2 * (1.0 / 7.0))))
    ln = e.astype(jnp.float32) * _LN2 + lnm
    return jnp.sign(d) * jnp.exp(BETA * ln)


def _sc_kernel(user_h, item_h, tbin_h, tday_h, cat_h, mean_ud_h, gm_h,
               WPIf_h, WPUf_h, BU_h, BI_h, WBITf_h, Alpha_h, AlphaUKf_h,
               WPUKTt_h, BTDay_h, BCU_h, WCU_h, out_h,
               uidx, iidx, tbv, widx, tday_v, cat_v, gm_v,
               btday_v, wcu_v, wpukt_v,
               bu_v, mu_v, al_v, bcu_v, bi_v, wbit_v,
               stag_v, dev_v, wpi_v, out_v, sem):
    wid = lax.axis_index("s") * NC + lax.axis_index("c")
    base = wid * BPW

    stage = []
    for c in range(NSEG):
        src = pl.ds(base + c * SEG, SEG)
        stage.append(pltpu.async_copy(user_h.at[src], uidx.at[c], sem))
        stage.append(pltpu.async_copy(item_h.at[src], iidx.at[c], sem))
        stage.append(pltpu.async_copy(tbin_h.at[src], tbv.at[c], sem))
    stage.append(pltpu.async_copy(tday_h.at[pl.ds(base, BPW)], tday_v, sem))
    stage.append(pltpu.async_copy(cat_h.at[pl.ds(base, BPW)], cat_v, sem))
    stage.append(pltpu.async_copy(gm_h, gm_v, sem))
    stage.append(pltpu.async_copy(BTDay_h, btday_v, sem))
    stage.append(pltpu.async_copy(WCU_h, wcu_v, sem))
    stage.append(pltpu.async_copy(WPUKTt_h, wpukt_v, sem))
    for cp in stage:
        cp.wait()

    for c in range(NSEG):
        for o in range(SEG // L):
            s = pl.ds(o * L, L)
            widx[c, s] = iidx[c, s] * N_F + tbv[c, s]

    copies = []
    for c in range(NSEG):
        dst = pl.ds(c * SEG, SEG)
        u, it, w = uidx.at[c], iidx.at[c], widx.at[c]
        copies.append(pltpu.async_copy(BU_h.at[u], bu_v.at[dst], sem))
        copies.append(pltpu.async_copy(mean_ud_h.at[u], mu_v.at[dst], sem))
        copies.append(pltpu.async_copy(Alpha_h.at[u], al_v.at[dst], sem))
        copies.append(pltpu.async_copy(BCU_h.at[u], bcu_v.at[dst], sem))
        copies.append(pltpu.async_copy(BI_h.at[it], bi_v.at[dst], sem))
        copies.append(pltpu.async_copy(WBITf_h.at[w], wbit_v.at[dst], sem))
        copies.append(pltpu.async_copy(WPUf_h.at[u], stag_v.at[dst], sem))
        copies.append(pltpu.async_copy(WPIf_h.at[it], wpi_v.at[dst], sem))
    for cp in copies:
        cp.wait()

    gm = gm_v[...]

    def chunk1(i, carry):
        s = pl.ds(i * L, L)
        d = tday_v[s].astype(jnp.float32) - mu_v[s]
        dev = _dev_t(d)
        dev_v[s] = dev
        cat16 = cat_v[s]
        butday = plsc.load_gather(btday_v, [cat16])
        cu_t = plsc.load_gather(wcu_v, [cat16])
        bias_user_time = bu_v[s] + al_v[s] * dev + butday
        bias_item_time = (bi_v[s] + wbit_v[s]) * (bcu_v[s] + cu_t)
        rows = i * L + lax.iota(jnp.int32, 16)
        acc = jnp.zeros((L,), jnp.float32)
        for j in range(N_F):
            cj = jnp.full((L,), j, jnp.int32)
            wpu_j = plsc.load_gather(stag_v, [rows, cj])
            wpi_j = plsc.load_gather(wpi_v, [rows, cj])
            pk_j = plsc.load_gather(wpukt_v, [cj, cat16])
            acc = acc + (wpu_j + pk_j) * wpi_j
        out_v[s] = gm + bias_user_time + bias_item_time + acc
        return carry

    lax.fori_loop(0, NCK, chunk1, 0)

    copies = []
    for c in range(NSEG):
        dst = pl.ds(c * SEG, SEG)
        copies.append(pltpu.async_copy(AlphaUKf_h.at[uidx.at[c]],
                                       stag_v.at[dst], sem))
    for cp in copies:
        cp.wait()

    def chunk2(i, carry):
        s = pl.ds(i * L, L)
        rows = i * L + lax.iota(jnp.int32, 16)
        acc = jnp.zeros((L,), jnp.float32)
        for j in range(N_F):
            cj = jnp.full((L,), j, jnp.int32)
            auk_j = plsc.load_gather(stag_v, [rows, cj])
            wpi_j = plsc.load_gather(wpi_v, [rows, cj])
            acc = acc + auk_j * wpi_j
        out_v[s] = out_v[s] + dev_v[s] * acc
        return carry

    lax.fori_loop(0, NCK, chunk2, 0)
    pltpu.sync_copy(out_v, out_h.at[pl.ds(base, BPW)])


def kernel(user, item, tbin, tday, maxday_cat, mean_ud, global_mean,
           WPI, WPU, BU, BI, WBIT, Alpha, AlphaUK, WPUKT, BTDay, BCU, WCU):
    WPUf = jnp.pad(WPU, ((0, 0), (0, 128 - N_F)))
    AUKf = jnp.pad(AlphaUK, ((0, 0), (0, 128 - N_F)))
    WPIf, WBITf = _detile([WPI.T, WBIT.T])

    mesh = plsc.VectorSubcoreMesh(core_axis_name="c", subcore_axis_name="s",
                                  num_cores=NC, num_subcores=NS)
    f32, i32 = jnp.float32, jnp.int32
    run = pl.kernel(
        _sc_kernel,
        out_type=jax.ShapeDtypeStruct((B,), f32),
        mesh=mesh,
        compiler_params=pltpu.CompilerParams(needs_layout_passes=False,
                                             use_tc_tiling_on_sc=False),
        scratch_types=[
            pltpu.VMEM((NSEG, SEG), i32),
            pltpu.VMEM((NSEG, SEG), i32),
            pltpu.VMEM((NSEG, SEG), i32),
            pltpu.VMEM((NSEG, SEG), i32),
            pltpu.VMEM((BPW,), i32),
            pltpu.VMEM((BPW,), i32),
            pltpu.VMEM((L,), f32),
            pltpu.VMEM((128,), f32),
            pltpu.VMEM((128,), f32),
            pltpu.VMEM((N_F, 128), f32),
            pltpu.VMEM((BPW,), f32),
            pltpu.VMEM((BPW,), f32),
            pltpu.VMEM((BPW,), f32),
            pltpu.VMEM((BPW,), f32),
            pltpu.VMEM((BPW,), f32),
            pltpu.VMEM((BPW,), f32),
            pltpu.VMEM((BPW, 128), f32),
            pltpu.VMEM((BPW,), f32),
            pltpu.VMEM((BPW, N_F), f32),
            pltpu.VMEM((BPW,), f32),
            pltpu.SemaphoreType.DMA,
        ],
    )
    return run(
        user, item, tbin, tday, maxday_cat, mean_ud,
        jnp.broadcast_to(global_mean, (L,)),
        WPIf.reshape(100000, N_F), WPUf,
        BU, BI, WBITf.reshape(100000 * N_F),
        Alpha, AUKf,
        WPUKT.T, BTDay, BCU, WCU)


# speedup vs baseline: 1.1314x; 1.1314x over previous
import functools

import jax
import jax.numpy as jnp
from jax import lax
from jax.experimental import pallas as pl
from jax.experimental.pallas import tpu as pltpu
from jax.experimental.pallas import tpu_sc as plsc

BETA = 0.4
B = 16384
N_F = 32
L = 16
NC, NS = 2, 16
NW = NC * NS
BPW = B // NW
NCK = BPW // L
NSEG = 4
SEG = BPW // NSEG
W = 8192

_LN2 = 0.6931471805599453


def _detile_body(nrows, *refs):
    half = len(refs) // 2
    ins, outs = refs[:half], refs[half:]
    for ref, oref, nr in zip(ins, outs, nrows):
        x = ref[...]
        if nr < N_F:
            x = jnp.concatenate(
                [x, jnp.zeros((N_F - nr, W), jnp.float32)], axis=0)
        t3 = x.T.reshape(W // 4, 4, N_F)
        oref[...] = jnp.concatenate([t3[:, a, :] for a in range(4)], axis=1)


def _detile(arrays_t):
    n = arrays_t[0].shape[1]
    nrows = tuple(a.shape[0] for a in arrays_t)
    no = n * N_F // 128
    return pl.pallas_call(
        functools.partial(_detile_body, nrows),
        grid=(pl.cdiv(n, W),),
        in_specs=[pl.BlockSpec((nr, W), lambda c: (0, c)) for nr in nrows],
        out_specs=[pl.BlockSpec((W // 4, 128), lambda c: (c, 0))
                   for _ in arrays_t],
        out_shape=[jax.ShapeDtypeStruct((no, 128), jnp.float32)
                   for _ in arrays_t],
    )(*arrays_t)


def _dev_t(d):
    ad = jnp.abs(d)
    bits = lax.bitcast_convert_type(ad, jnp.int32)
    e = (bits >> 23) - 127
    m = lax.bitcast_convert_type((bits & 0x7FFFFF) | 0x3F800000, jnp.float32)
    z = (m - 1.0) / (m + 1.0)
    z2 = z * z
    lnm = 2.0 * z * (1.0 + z2 * (1.0 / 3.0 + z2 * (0.2 + z2 * (1.0 / 7.0))))
    ln = e.astype(jnp.float32) * _LN2 + lnm
    return jnp.sign(d) * jnp.exp(BETA * ln)


def _sc_kernel(user_h, item_h, tbin_h, tday_h, cat_h, mean_ud_h, gm_h,
               WPIf_h, WPUf_h, BU_h, BI_h, WBITf_h, Alpha_h, AlphaUKf_h,
               WPUKTt_h, BTDay_h, BCU_h, WCU_h, out_h,
               uidx, iidx, tbv, widx, tday_v, cat_v, gm_v,
               btday_v, wcu_v, wpukt_v,
               bu_v, mu_v, al_v, bcu_v, bi_v, wbit_v,
               wpu_v, auk_v, wpi_v, out_v, sem):
    wid = lax.axis_index("s") * NC + lax.axis_index("c")
    base = wid * BPW

    stage = []
    for c in range(NSEG):
        src = pl.ds(base + c * SEG, SEG)
        stage.append(pltpu.async_copy(user_h.at[src], uidx.at[c], sem))
        stage.append(pltpu.async_copy(item_h.at[src], iidx.at[c], sem))
        stage.append(pltpu.async_copy(tbin_h.at[src], tbv.at[c], sem))
    stage.append(pltpu.async_copy(tday_h.at[pl.ds(base, BPW)], tday_v, sem))
    stage.append(pltpu.async_copy(cat_h.at[pl.ds(base, BPW)], cat_v, sem))
    stage.append(pltpu.async_copy(gm_h, gm_v, sem))
    stage.append(pltpu.async_copy(BTDay_h, btday_v, sem))
    stage.append(pltpu.async_copy(WCU_h, wcu_v, sem))
    stage.append(pltpu.async_copy(WPUKTt_h, wpukt_v, sem))
    for cp in stage:
        cp.wait()

    for c in range(NSEG):
        for o in range(SEG // L):
            s = pl.ds(o * L, L)
            widx[c, s] = iidx[c, s] * N_F + tbv[c, s]

    copies = []
    for c in range(NSEG):
        dst = pl.ds(c * SEG, SEG)
        u, it, w = uidx.at[c], iidx.at[c], widx.at[c]
        copies.append(pltpu.async_copy(BU_h.at[u], bu_v.at[dst], sem))
        copies.append(pltpu.async_copy(mean_ud_h.at[u], mu_v.at[dst], sem))
        copies.append(pltpu.async_copy(Alpha_h.at[u], al_v.at[dst], sem))
        copies.append(pltpu.async_copy(BCU_h.at[u], bcu_v.at[dst], sem))
        copies.append(pltpu.async_copy(BI_h.at[it], bi_v.at[dst], sem))
        copies.append(pltpu.async_copy(WBITf_h.at[w], wbit_v.at[dst], sem))
        copies.append(pltpu.async_copy(WPUf_h.at[u], wpu_v.at[dst], sem))
        copies.append(pltpu.async_copy(AlphaUKf_h.at[u], auk_v.at[dst], sem))
        copies.append(pltpu.async_copy(WPIf_h.at[it], wpi_v.at[dst], sem))
    for cp in copies:
        cp.wait()

    gm = gm_v[...]

    def chunk(i, carry):
        s = pl.ds(i * L, L)
        d = tday_v[s].astype(jnp.float32) - mu_v[s]
        dev = _dev_t(d)
        cat16 = cat_v[s]
        butday = plsc.load_gather(btday_v, [cat16])
        cu_t = plsc.load_gather(wcu_v, [cat16])
        bias_user_time = bu_v[s] + al_v[s] * dev + butday
        bias_item_time = (bi_v[s] + wbit_v[s]) * (bcu_v[s] + cu_t)
        rows = i * L + lax.iota(jnp.int32, 16)
        acc = jnp.zeros((L,), jnp.float32)
        for j in range(N_F):
            cj = jnp.full((L,), j, jnp.int32)
            wpu_j = plsc.load_gather(wpu_v, [rows, cj])
            auk_j = plsc.load_gather(auk_v, [rows, cj])
            wpi_j = plsc.load_gather(wpi_v, [rows, cj])
            pk_j = plsc.load_gather(wpukt_v, [cj, cat16])
            acc = acc + (wpu_j + auk_j * dev + pk_j) * wpi_j
        out_v[s] = gm + bias_user_time + bias_item_time + acc
        return carry

    lax.fori_loop(0, NCK, chunk, 0)
    pltpu.sync_copy(out_v, out_h.at[pl.ds(base, BPW)])


def kernel(user, item, tbin, tday, maxday_cat, mean_ud, global_mean,
           WPI, WPU, BU, BI, WBIT, Alpha, AlphaUK, WPUKT, BTDay, BCU, WCU):
    WPUf = jnp.pad(WPU, ((0, 0), (0, 128 - N_F)))
    (AUKf,) = _detile([AlphaUK.T])
    WPIf, WBITf = _detile([WPI.T, WBIT.T])

    mesh = plsc.VectorSubcoreMesh(core_axis_name="c", subcore_axis_name="s",
                                  num_cores=NC, num_subcores=NS)
    f32, i32 = jnp.float32, jnp.int32
    run = pl.kernel(
        _sc_kernel,
        out_type=jax.ShapeDtypeStruct((B,), f32),
        mesh=mesh,
        compiler_params=pltpu.CompilerParams(needs_layout_passes=False,
                                             use_tc_tiling_on_sc=False),
        scratch_types=[
            pltpu.VMEM((NSEG, SEG), i32),
            pltpu.VMEM((NSEG, SEG), i32),
            pltpu.VMEM((NSEG, SEG), i32),
            pltpu.VMEM((NSEG, SEG), i32),
            pltpu.VMEM((BPW,), i32),
            pltpu.VMEM((BPW,), i32),
            pltpu.VMEM((L,), f32),
            pltpu.VMEM((128,), f32),
            pltpu.VMEM((128,), f32),
            pltpu.VMEM((N_F, 128), f32),
            pltpu.VMEM((BPW,), f32),
            pltpu.VMEM((BPW,), f32),
            pltpu.VMEM((BPW,), f32),
            pltpu.VMEM((BPW,), f32),
            pltpu.VMEM((BPW,), f32),
            pltpu.VMEM((BPW,), f32),
            pltpu.VMEM((BPW, 128), f32),
            pltpu.VMEM((BPW, N_F), f32),
            pltpu.VMEM((BPW, N_F), f32),
            pltpu.VMEM((BPW,), f32),
            pltpu.SemaphoreType.DMA,
        ],
    )
    return run(
        user, item, tbin, tday, maxday_cat, mean_ud,
        jnp.broadcast_to(global_mean, (L,)),
        WPIf.reshape(100000, N_F), WPUf,
        BU, BI, WBITf.reshape(100000 * N_F),
        Alpha, AUKf.reshape(1000000, N_F),
        WPUKT.T, BTDay, BCU, WCU)
